# baseline (speedup 1.0000x reference)
import jax
import jax.numpy as jnp
import numpy as np
from jax import lax
from jax.experimental import pallas as pl
from jax.experimental.pallas import tpu as pltpu
from jax.experimental.pallas import tpu_sc as plsc

B = 1024
M = 10
S = 20
H = 224
W = 224
NHID = 100
NOUT = 10
LOG2PI = np.float32(np.log(2.0 * np.pi))


def _rand_consts():
    skey = jax.random.key(1234)
    gs, zs = [], []
    for i in range(S):
        k1, k2 = jax.random.split(jax.random.fold_in(skey, i))
        gs.append(jax.random.gumbel(k1, (B, M), jnp.float32))
        zs.append(jax.random.normal(k2, (B, 2), jnp.float32))
    g = jnp.stack(gs).transpose(2, 0, 1)
    z = jnp.stack(zs).transpose(2, 0, 1)
    return g, z


def _sample_body(mg_ref, g_ref, z_ref, idx_ref, cidx_ref, loc0_ref, loc1_ref,
                 flp_ref):
    biota1 = lax.broadcasted_iota(jnp.int32, (1, B), 1)
    pis = [mg_ref[0, m] for m in range(M)]
    pv = []
    for j in range(M):
        compj = (biota1 * M + j) >> 10
        acc = jnp.full((1, B), pis[0])
        for m in range(1, M):
            acc = jnp.where(compj == m, pis[m], acc)
        pv.append(acc)
    mx = pv[0]
    for j in range(1, M):
        mx = jnp.maximum(mx, pv[j])
    es = [jnp.exp(p - mx) for p in pv]
    ssum = es[0]
    for j in range(1, M):
        ssum = ssum + es[j]
    lgs = [jnp.log(e / ssum) for e in es]

    best = lgs[0] + g_ref[0]
    jidx = jnp.zeros((S, B), jnp.int32)
    for j in range(1, M):
        sc = lgs[j] + g_ref[j]
        take = sc > best
        best = jnp.where(take, sc, best)
        jidx = jnp.where(take, j, jidx)
    biota2 = lax.broadcasted_iota(jnp.int32, (S, B), 1)
    comp = (biota2 * M + jidx) >> 10

    def sel(row):
        vals = [mg_ref[row, m] for m in range(M)]
        acc = jnp.full((S, B), vals[0])
        for m in range(1, M):
            acc = jnp.where(comp == m, vals[m], acc)
        return acc

    mux = sel(1)
    muy = sel(2)
    sx = jnp.exp(sel(3))
    sy = jnp.exp(sel(4))
    r = jnp.tanh(sel(5))
    a01 = r * sx * sy
    l00 = jnp.sqrt(sx * sx)
    l10 = a01 / l00
    l11 = jnp.sqrt(sy * sy - l10 * l10)
    z0 = z_ref[0]
    z1 = z_ref[1]
    xs0 = mux + l00 * z0
    xs1 = muy + (l10 * z0 + l11 * z1)
    d0 = xs0 - mux
    d1 = xs1 - muy
    y0 = d0 / l00
    y1 = (d1 - l10 * y0) / l11
    logprob = (-0.5 * (y0 * y0 + y1 * y1) - jnp.log(l00) - jnp.log(l11)
               - LOG2PI)
    loc0 = jnp.tanh(xs0)
    loc1 = jnp.tanh(xs1)
    di0 = (0.5 * ((loc0 + 1.0) * H) - 0.1).astype(jnp.int32)
    di1 = (0.5 * ((loc1 + 1.0) * W) - 0.1).astype(jnp.int32)
    idx_ref[...] = biota2 * H + di0
    cidx_ref[...] = di1
    loc0_ref[...] = loc0
    loc1_ref[...] = loc1
    flp_ref[...] = jnp.sum(logprob, axis=0, keepdims=True)


def _run_sample(mg, g, z):
    return pl.pallas_call(
        _sample_body,
        out_shape=(
            jax.ShapeDtypeStruct((S, B), jnp.int32),
            jax.ShapeDtypeStruct((S, B), jnp.int32),
            jax.ShapeDtypeStruct((S, B), jnp.float32),
            jax.ShapeDtypeStruct((S, B), jnp.float32),
            jax.ShapeDtypeStruct((1, B), jnp.float32),
        ),
        in_specs=[
            pl.BlockSpec(memory_space=pltpu.SMEM),
            pl.BlockSpec(memory_space=pltpu.VMEM),
            pl.BlockSpec(memory_space=pltpu.VMEM),
        ],
    )(mg, g, z)


def _sc_gather(x2d, ridx, cidx):
    info = plsc.get_sparse_core_info()
    nc, ns, nl = info.num_cores, info.num_subcores, info.num_lanes
    nw = nc * ns
    chunk = (S * B) // nw
    half = chunk // 2
    mesh = plsc.VectorSubcoreMesh(core_axis_name="c", subcore_axis_name="s")

    def body(x_hbm, ridx_hbm, cidx_hbm, out_hbm, ridx_v, cidx_v, rows_v,
             out_v, sem):
        wid = lax.axis_index("s") * nc + lax.axis_index("c")
        base = wid * chunk
        pltpu.sync_copy(ridx_hbm.at[pl.ds(base, chunk)], ridx_v)
        pltpu.sync_copy(cidx_hbm.at[pl.ds(base, chunk)], cidx_v)
        for hh in range(2):
            pltpu.async_copy(x_hbm.at[ridx_v.at[pl.ds(hh * half, half)]],
                             rows_v, sem).wait()
            for grp in range(half // nl):
                rows16 = lax.iota(jnp.int32, nl) + grp * nl
                cols16 = cidx_v[pl.ds(hh * half + grp * nl, nl)]
                out_v[pl.ds(hh * half + grp * nl, nl)] = plsc.load_gather(
                    rows_v, [rows16, cols16])
        pltpu.sync_copy(out_v, out_hbm.at[pl.ds(base, chunk)])

    return pl.kernel(
        body,
        mesh=mesh,
        out_type=jax.ShapeDtypeStruct((S * B,), jnp.float32),
        compiler_params=pltpu.CompilerParams(use_tc_tiling_on_sc=False,
                                             needs_layout_passes=False),
        scratch_types=[
            pltpu.VMEM((chunk,), jnp.int32),
            pltpu.VMEM((chunk,), jnp.int32),
            pltpu.VMEM((half, W), jnp.float32),
            pltpu.VMEM((chunk,), jnp.float32),
            pltpu.SemaphoreType.DMA,
        ],
    )(x2d, ridx, cidx)


def _gcn_body(pix_ref, l0_ref, l1_ref, w1_ref, b1_ref, w2_ref, b2_ref,
              fwt_ref, fb_ref, o_ref):
    w1 = w1_ref[...]
    hp = None
    for i in range(S):
        feat_i = jnp.concatenate(
            [pix_ref[:, i:i + 1], l0_ref[:, i:i + 1], l1_ref[:, i:i + 1]],
            axis=1)
        y = jnp.dot(feat_i, w1, preferred_element_type=jnp.float32)
        yb = y.astype(jnp.bfloat16).astype(jnp.float32)
        hp = yb if hp is None else hp + yb
    h = jnp.maximum(hp + b1_ref[...], 0.0)
    t = jnp.dot(h, w2_ref[...], preferred_element_type=jnp.float32)
    h2 = 20.0 * t.astype(jnp.bfloat16).astype(jnp.float32) + b2_ref[...]
    mx0 = jnp.max(h2, axis=0, keepdims=True)
    sh = h2 - mx0
    g = sh - jnp.log(jnp.sum(jnp.exp(sh), axis=0, keepdims=True))
    gfull = jnp.concatenate([g] * S, axis=1)
    fin = (jnp.dot(gfull, fwt_ref[...], preferred_element_type=jnp.float32)
           + fb_ref[...])
    mx1 = jnp.max(fin, axis=1, keepdims=True)
    ef = jnp.exp(fin - mx1)
    o_ref[...] = ef / jnp.sum(ef, axis=1, keepdims=True)


def _run_gcn(pix, l0, l1, gc1_w, gc1_b, gc2_w, gc2_b, fcf_wt, fcf_b):
    return pl.pallas_call(
        _gcn_body,
        out_shape=jax.ShapeDtypeStruct((B, NOUT), jnp.float32),
    )(pix, l0, l1, gc1_w, gc1_b, gc2_w, gc2_b, fcf_wt, fcf_b)


def kernel(x, m_g_params, gc1_w, gc1_b, gc2_w, gc2_b, fcf_w, fcf_b):
    mg = m_g_params.reshape(M, 6).T
    g, z = _rand_consts()
    idx, cidx, loc0, loc1, flp = _run_sample(mg, g, z)
    pix = _sc_gather(x.reshape(B * H, W), idx.reshape(-1), cidx.reshape(-1))
    o = _run_gcn(pix.reshape(B, S), loc0.reshape(B, S), loc1.reshape(B, S),
                 gc1_w, gc1_b.reshape(1, NHID), gc2_w,
                 gc2_b.reshape(1, NOUT), fcf_w.T, fcf_b.reshape(1, NOUT))
    return o, flp.reshape(B)


# speedup vs baseline: 1.1858x; 1.1858x over previous
import jax
import jax.numpy as jnp
import numpy as np
from jax import lax
from jax.experimental import pallas as pl
from jax.experimental.pallas import tpu as pltpu
from jax.experimental.pallas import tpu_sc as plsc

B = 1024
M = 10
S = 20
H = 224
W = 224
NHID = 100
NOUT = 10
LOG2PI = np.float32(np.log(2.0 * np.pi))


def _make_rand_consts():
    skey = jax.random.key(1234)
    gs, zs = [], []
    for i in range(S):
        k1, k2 = jax.random.split(jax.random.fold_in(skey, i))
        gs.append(jax.random.gumbel(k1, (B, M), jnp.float32))
        zs.append(jax.random.normal(k2, (B, 2), jnp.float32))
    g = np.ascontiguousarray(np.asarray(jnp.stack(gs)).transpose(2, 0, 1))
    z = np.ascontiguousarray(np.asarray(jnp.stack(zs)).transpose(2, 0, 1))
    return g, z


_G_CONST, _Z_CONST = _make_rand_consts()


def _sample_body(mg_ref, g_ref, z_ref, idx_ref, cidx_ref, loc0_ref, loc1_ref,
                 flp_ref):
    biota1 = lax.broadcasted_iota(jnp.int32, (1, B), 1)
    pis = [mg_ref[0, m] for m in range(M)]
    pv = []
    for j in range(M):
        compj = (biota1 * M + j) >> 10
        acc = jnp.full((1, B), pis[0])
        for m in range(1, M):
            acc = jnp.where(compj == m, pis[m], acc)
        pv.append(acc)
    mx = pv[0]
    for j in range(1, M):
        mx = jnp.maximum(mx, pv[j])
    es = [jnp.exp(p - mx) for p in pv]
    ssum = es[0]
    for j in range(1, M):
        ssum = ssum + es[j]
    lgs = [jnp.log(e / ssum) for e in es]

    best = lgs[0] + g_ref[0]
    jidx = jnp.zeros((S, B), jnp.int32)
    for j in range(1, M):
        sc = lgs[j] + g_ref[j]
        take = sc > best
        best = jnp.where(take, sc, best)
        jidx = jnp.where(take, j, jidx)
    biota2 = lax.broadcasted_iota(jnp.int32, (S, B), 1)
    comp = (biota2 * M + jidx) >> 10

    def sel(row):
        vals = [mg_ref[row, m] for m in range(M)]
        acc = jnp.full((S, B), vals[0])
        for m in range(1, M):
            acc = jnp.where(comp == m, vals[m], acc)
        return acc

    mux = sel(1)
    muy = sel(2)
    sx = jnp.exp(sel(3))
    sy = jnp.exp(sel(4))
    r = jnp.tanh(sel(5))
    a01 = r * sx * sy
    l00 = jnp.sqrt(sx * sx)
    l10 = a01 / l00
    l11 = jnp.sqrt(sy * sy - l10 * l10)
    z0 = z_ref[0]
    z1 = z_ref[1]
    xs0 = mux + l00 * z0
    xs1 = muy + (l10 * z0 + l11 * z1)
    d0 = xs0 - mux
    d1 = xs1 - muy
    y0 = d0 / l00
    y1 = (d1 - l10 * y0) / l11
    logprob = (-0.5 * (y0 * y0 + y1 * y1) - jnp.log(l00) - jnp.log(l11)
               - LOG2PI)
    loc0 = jnp.tanh(xs0)
    loc1 = jnp.tanh(xs1)
    di0 = (0.5 * ((loc0 + 1.0) * H) - 0.1).astype(jnp.int32)
    di1 = (0.5 * ((loc1 + 1.0) * W) - 0.1).astype(jnp.int32)
    idx_ref[...] = biota2 * H + di0
    cidx_ref[...] = di1
    loc0_ref[...] = loc0
    loc1_ref[...] = loc1
    flp_ref[...] = jnp.sum(logprob, axis=0, keepdims=True)


def _run_sample(mg, g, z):
    return pl.pallas_call(
        _sample_body,
        out_shape=(
            jax.ShapeDtypeStruct((S, B), jnp.int32),
            jax.ShapeDtypeStruct((S, B), jnp.int32),
            jax.ShapeDtypeStruct((S, B), jnp.float32),
            jax.ShapeDtypeStruct((S, B), jnp.float32),
            jax.ShapeDtypeStruct((1, B), jnp.float32),
        ),
        in_specs=[
            pl.BlockSpec(memory_space=pltpu.SMEM),
            pl.BlockSpec(memory_space=pltpu.VMEM),
            pl.BlockSpec(memory_space=pltpu.VMEM),
        ],
    )(mg, g, z)


def _sc_gather(x2d, ridx, cidx):
    info = plsc.get_sparse_core_info()
    nc, ns, nl = info.num_cores, info.num_subcores, info.num_lanes
    nw = nc * ns
    chunk = (S * B) // nw
    half = chunk // 2
    mesh = plsc.VectorSubcoreMesh(core_axis_name="c", subcore_axis_name="s")

    def body(x_hbm, ridx_hbm, cidx_hbm, out_hbm, ridx_v, cidx_v, rows_v,
             out_v, sem):
        wid = lax.axis_index("s") * nc + lax.axis_index("c")
        base = wid * chunk
        pltpu.sync_copy(ridx_hbm.at[pl.ds(base, chunk)], ridx_v)
        pltpu.sync_copy(cidx_hbm.at[pl.ds(base, chunk)], cidx_v)
        for hh in range(2):
            pltpu.async_copy(x_hbm.at[ridx_v.at[pl.ds(hh * half, half)]],
                             rows_v, sem).wait()
            for grp in range(half // nl):
                rows16 = lax.iota(jnp.int32, nl) + grp * nl
                cols16 = cidx_v[pl.ds(hh * half + grp * nl, nl)]
                out_v[pl.ds(hh * half + grp * nl, nl)] = plsc.load_gather(
                    rows_v, [rows16, cols16])
        pltpu.sync_copy(out_v, out_hbm.at[pl.ds(base, chunk)])

    return pl.kernel(
        body,
        mesh=mesh,
        out_type=jax.ShapeDtypeStruct((S * B,), jnp.float32),
        compiler_params=pltpu.CompilerParams(use_tc_tiling_on_sc=False,
                                             needs_layout_passes=False),
        scratch_types=[
            pltpu.VMEM((chunk,), jnp.int32),
            pltpu.VMEM((chunk,), jnp.int32),
            pltpu.VMEM((half, W), jnp.float32),
            pltpu.VMEM((chunk,), jnp.float32),
            pltpu.SemaphoreType.DMA,
        ],
    )(x2d, ridx, cidx)


def _gcn_body(pix_ref, l0_ref, l1_ref, w1_ref, b1_ref, w2_ref, b2_ref,
              fwt_ref, fb_ref, o_ref):
    w1 = w1_ref[...]
    hp = None
    for i in range(S):
        feat_i = jnp.concatenate(
            [pix_ref[:, i:i + 1], l0_ref[:, i:i + 1], l1_ref[:, i:i + 1]],
            axis=1)
        y = jnp.dot(feat_i, w1, preferred_element_type=jnp.float32)
        yb = y.astype(jnp.bfloat16).astype(jnp.float32)
        hp = yb if hp is None else hp + yb
    h = jnp.maximum(hp + b1_ref[...], 0.0)
    t = jnp.dot(h, w2_ref[...], preferred_element_type=jnp.float32)
    h2 = 20.0 * t.astype(jnp.bfloat16).astype(jnp.float32) + b2_ref[...]
    mx0 = jnp.max(h2, axis=0, keepdims=True)
    sh = h2 - mx0
    g = sh - jnp.log(jnp.sum(jnp.exp(sh), axis=0, keepdims=True))
    gfull = jnp.concatenate([g] * S, axis=1)
    fin = (jnp.dot(gfull, fwt_ref[...], preferred_element_type=jnp.float32)
           + fb_ref[...])
    mx1 = jnp.max(fin, axis=1, keepdims=True)
    ef = jnp.exp(fin - mx1)
    o_ref[...] = ef / jnp.sum(ef, axis=1, keepdims=True)


def _run_gcn(pix, l0, l1, gc1_w, gc1_b, gc2_w, gc2_b, fcf_wt, fcf_b):
    return pl.pallas_call(
        _gcn_body,
        out_shape=jax.ShapeDtypeStruct((B, NOUT), jnp.float32),
    )(pix, l0, l1, gc1_w, gc1_b, gc2_w, gc2_b, fcf_wt, fcf_b)


def kernel(x, m_g_params, gc1_w, gc1_b, gc2_w, gc2_b, fcf_w, fcf_b):
    mg = m_g_params.reshape(M, 6).T
    g = jnp.asarray(_G_CONST)
    z = jnp.asarray(_Z_CONST)
    idx, cidx, loc0, loc1, flp = _run_sample(mg, g, z)
    pix = _sc_gather(x.reshape(B * H, W), idx.reshape(-1), cidx.reshape(-1))
    o = _run_gcn(pix.reshape(B, S), loc0.reshape(B, S), loc1.reshape(B, S),
                 gc1_w, gc1_b.reshape(1, NHID), gc2_w,
                 gc2_b.reshape(1, NOUT), fcf_w.T, fcf_b.reshape(1, NOUT))
    return o, flp.reshape(B)
